# baseline (speedup 1.0000x reference)
import functools

import jax
import jax.numpy as jnp
from jax import lax
from jax.experimental import pallas as pl
from jax.experimental.pallas import tpu as pltpu
from jax.experimental.pallas import tpu_sc as plsc

B = 4096
L = 200
D = 64
NC = 2
NS = 16
NW = NC * NS
BPW = B // NW
NCH = 2
CH = L // NCH
LFULL = (L // 16) * 16
NTAIL = D // 2


def _body(idx_hbm, table_hbm, out_hbm, idx_v, rows_v, out_v, sem):
    wid = lax.axis_index("s") * NC + lax.axis_index("c")
    iota = lax.iota(jnp.int32, 16)
    lane_lo = iota < 8
    tail_l = jnp.where(lane_lo, 192 + iota, 184 + iota)
    tail_d_off = jnp.where(lane_lo, 0, 1)

    def per_row(i, _):
        b = wid * BPW + i
        pltpu.sync_copy(idx_hbm.at[b], idx_v)
        descs = [
            pltpu.async_copy(
                table_hbm.at[idx_v.at[j]],
                rows_v.at[pl.ds(j * CH, CH)],
                sem,
            )
            for j in range(NCH)
        ]
        for dsc in descs:
            dsc.wait()

        def per_d(d, _):
            d_vec = jnp.full((16,), 0, jnp.int32) + d
            for l0 in range(LFULL // 16):
                l_vec = l0 * 16 + iota
                v = plsc.load_gather(rows_v, [l_vec, d_vec])
                plsc.store_scatter(out_v, [d_vec, l_vec], v)
            return 0

        lax.fori_loop(0, D, per_d, 0, unroll=False)

        def per_tail(p, _):
            d_vec = 2 * p + tail_d_off
            v = plsc.load_gather(rows_v, [tail_l, d_vec])
            plsc.store_scatter(out_v, [d_vec, tail_l], v)
            return 0

        lax.fori_loop(0, NTAIL, per_tail, 0, unroll=False)
        pltpu.sync_copy(out_v, out_hbm.at[b])
        return 0

    lax.fori_loop(0, BPW, per_row, 0, unroll=False)


@functools.partial(jax.jit, static_argnames=())
def kernel(input, table):
    idx3 = input.astype(jnp.int32).reshape(B, NCH, CH)
    run = pl.kernel(
        _body,
        out_type=jax.ShapeDtypeStruct((B, D, L), jnp.float32),
        mesh=plsc.VectorSubcoreMesh(core_axis_name="c", subcore_axis_name="s"),
        compiler_params=pltpu.CompilerParams(
            needs_layout_passes=False, use_tc_tiling_on_sc=False
        ),
        scratch_types=[
            pltpu.VMEM((NCH, CH), jnp.int32),
            pltpu.VMEM((L, D), jnp.float32),
            pltpu.VMEM((D, L), jnp.float32),
            pltpu.SemaphoreType.DMA,
        ],
    )
    return run(idx3, table)


# speedup vs baseline: 1.4475x; 1.4475x over previous
import functools

import jax
import jax.numpy as jnp
from jax import lax
from jax.experimental import pallas as pl
from jax.experimental.pallas import tpu as pltpu
from jax.experimental.pallas import tpu_sc as plsc

B = 4096
L = 200
D = 64
NC = 2
NS = 16
NW = NC * NS
BPW = B // NW
NCH = 2
CH = L // NCH
LFULL = (L // 16) * 16
NTAIL = D // 2
NBUF = 4
NOUT = 2


def _body(idx_hbm, table_hbm, out_hbm, idx_all, rows, outb, *sems):
    gsems = sems[:NBUF]
    osems = sems[NBUF:]
    wid = lax.axis_index("s") * NC + lax.axis_index("c")
    iota = lax.iota(jnp.int32, 16)
    lane_lo = iota < 8
    tail_l = jnp.where(lane_lo, 192 + iota, 184 + iota)
    tail_d_off = jnp.where(lane_lo, 0, 1)
    l_vecs = [l0 * 16 + iota for l0 in range(LFULL // 16)]

    pltpu.sync_copy(idx_hbm.at[wid], idx_all)

    def fire_gather(i, p):
        for j in range(NCH):
            pltpu.async_copy(
                table_hbm.at[idx_all.at[i, j]],
                rows.at[p, pl.ds(j * CH, CH)],
                gsems[p],
            )

    def wait_gather(p):
        pltpu.make_async_copy(
            table_hbm.at[pl.ds(0, L)], rows.at[p], gsems[p]
        ).wait()

    def wait_out(q):
        pltpu.make_async_copy(outb.at[q], out_hbm.at[0], osems[q]).wait()

    def transpose(p, q):
        @plsc.parallel_loop(0, D)
        def per_d(d):
            d_vec = jnp.full((16,), 0, jnp.int32) + d
            for l_vec in l_vecs:
                v = plsc.load_gather(rows.at[p], [l_vec, d_vec])
                plsc.store_scatter(outb.at[q], [d_vec, l_vec], v)

        @plsc.parallel_loop(0, NTAIL)
        def per_tail(t):
            d_vec = 2 * t + tail_d_off
            v = plsc.load_gather(rows.at[p], [tail_l, d_vec])
            plsc.store_scatter(outb.at[q], [d_vec, tail_l], v)

    for i in range(NBUF - 1):
        fire_gather(i, i % NBUF)

    def step(k, _):
        for s in range(NBUF):
            i = NBUF * k + s
            q = s % NOUT
            wait_gather(s)
            inext = i + (NBUF - 1)

            @pl.when(inext < BPW)
            def _():
                fire_gather(inext, (s - 1) % NBUF)

            @pl.when(i >= NOUT)
            def _():
                wait_out(q)

            transpose(s, q)
            pltpu.async_copy(outb.at[q], out_hbm.at[wid * BPW + i], osems[q])
        return 0

    lax.fori_loop(0, BPW // NBUF, step, 0, unroll=False)
    wait_out(0)
    wait_out(1)


@functools.partial(jax.jit, static_argnames=())
def kernel(input, table):
    idx4 = input.astype(jnp.int32).reshape(NW, BPW, NCH, CH)
    run = pl.kernel(
        _body,
        out_type=jax.ShapeDtypeStruct((B, D, L), jnp.float32),
        mesh=plsc.VectorSubcoreMesh(core_axis_name="c", subcore_axis_name="s"),
        compiler_params=pltpu.CompilerParams(
            needs_layout_passes=False, use_tc_tiling_on_sc=False
        ),
        scratch_types=[
            pltpu.VMEM((BPW, NCH, CH), jnp.int32),
            pltpu.VMEM((NBUF, L, D), jnp.float32),
            pltpu.VMEM((NOUT, D, L), jnp.float32),
        ]
        + [pltpu.SemaphoreType.DMA] * (NBUF + NOUT),
    )
    return run(idx4, table)


# speedup vs baseline: 1.9654x; 1.3578x over previous
import functools

import jax
import jax.numpy as jnp
from jax import lax
from jax.experimental import pallas as pl
from jax.experimental.pallas import tpu as pltpu
from jax.experimental.pallas import tpu_sc as plsc

B = 4096
L = 200
D = 64
NC = 2
NS = 16
NW = NC * NS
BPW = B // NW
NCH = 2
CH = L // NCH
LP = 201
NBUF = 4
NOUT = 2


def _body(idx_hbm, table_hbm, out_hbm, idx_all, rows, outb, *sems):
    gsems = sems[:NBUF]
    osems = sems[NBUF:]
    wid = lax.axis_index("s") * NC + lax.axis_index("c")
    iota = lax.iota(jnp.int32, 16)
    d_vecs = [q4 * 16 + iota for q4 in range(D // 16)]

    pltpu.sync_copy(idx_hbm.at[wid], idx_all)

    def fire_gather(i, p):
        for j in range(NCH):
            pltpu.async_copy(
                table_hbm.at[idx_all.at[i, j]],
                rows.at[p, pl.ds(j * CH, CH)],
                gsems[p],
            )

    def wait_gather(p):
        pltpu.make_async_copy(
            table_hbm.at[pl.ds(0, L)], rows.at[p], gsems[p]
        ).wait()

    def wait_out(q):
        pltpu.make_async_copy(
            outb.at[q, :, pl.ds(0, L)], out_hbm.at[0], osems[q]
        ).wait()

    def transpose(p, q):
        @plsc.parallel_loop(0, L)
        def per_l(l):
            l_vec = jnp.full((16,), 0, jnp.int32) + l
            for q4 in range(D // 16):
                v = rows[p, l, pl.ds(q4 * 16, 16)]
                plsc.store_scatter(outb.at[q], [d_vecs[q4], l_vec], v)

    for i in range(NBUF - 1):
        fire_gather(i, i % NBUF)

    def step(k, _):
        for s in range(NBUF):
            i = NBUF * k + s
            q = s % NOUT
            wait_gather(s)
            inext = i + (NBUF - 1)

            @pl.when(inext < BPW)
            def _():
                fire_gather(inext, (s - 1) % NBUF)

            @pl.when(i >= NOUT)
            def _():
                wait_out(q)

            transpose(s, q)
            pltpu.async_copy(
                outb.at[q, :, pl.ds(0, L)],
                out_hbm.at[wid * BPW + i],
                osems[q],
            )
        return 0

    lax.fori_loop(0, BPW // NBUF, step, 0, unroll=False)
    wait_out(0)
    wait_out(1)


@functools.partial(jax.jit, static_argnames=())
def kernel(input, table):
    idx4 = input.astype(jnp.int32).reshape(NW, BPW, NCH, CH)
    run = pl.kernel(
        _body,
        out_type=jax.ShapeDtypeStruct((B, D, L), jnp.float32),
        mesh=plsc.VectorSubcoreMesh(core_axis_name="c", subcore_axis_name="s"),
        compiler_params=pltpu.CompilerParams(
            needs_layout_passes=False, use_tc_tiling_on_sc=False
        ),
        scratch_types=[
            pltpu.VMEM((BPW, NCH, CH), jnp.int32),
            pltpu.VMEM((NBUF, L, D), jnp.float32),
            pltpu.VMEM((NOUT, D, LP), jnp.float32),
        ]
        + [pltpu.SemaphoreType.DMA] * (NBUF + NOUT),
    )
    return run(idx4, table)


# speedup vs baseline: 4.2741x; 2.1747x over previous
import functools

import jax
import jax.numpy as jnp
from jax import lax
from jax.experimental import pallas as pl
from jax.experimental.pallas import tpu as pltpu
from jax.experimental.pallas import tpu_sc as plsc

B = 4096
L = 200
D = 64
V = 1000000
NC = 2
NS = 16
NW = NC * NS
RP = 129
VBT = (V + 127) // 128
VB_PER_W = VBT // NW
VB_EXTRA = VBT % NW
SROWS = VBT * 64
K1_STEPS = (VB_PER_W + 2) // 2
NT = L // 8


def _relayout_body(tbl_hbm, scr_hbm, blk, rp1, stage, *sems):
    gsems = sems[:2]
    osems = sems[2:]
    wid = lax.axis_index("s") * NC + lax.axis_index("c")
    iota = lax.iota(jnp.int32, 16)
    chunk16 = [iota + 16 * c for c in range(8)]
    iota_rp_off = [(16 * (p % 4) + iota) * RP + (p // 4) for p in range(8)]

    n_full = VB_PER_W + jnp.where(wid < VB_EXTRA, 1, 0)
    base = wid * VB_PER_W + jnp.minimum(wid, VB_EXTRA)

    def fire_in(u, p):
        off = pl.multiple_of(u * 128, 128)
        pltpu.async_copy(tbl_hbm.at[:, pl.ds(off, 128)], blk.at[p], gsems[p])

    def wait_in(p):
        pltpu.make_async_copy(
            tbl_hbm.at[:, pl.ds(0, 128)], blk.at[p], gsems[p]
        ).wait()

    def wait_out(q):
        pltpu.make_async_copy(
            stage.at[q], scr_hbm.at[pl.ds(0, D), :], osems[q]
        ).wait()

    def transpose_block(p, q):
        @plsc.parallel_loop(0, D)
        def _(d):
            db = jnp.full((16,), 0, jnp.int32) + d * RP
            for c8 in range(8):
                v = blk[p, d, pl.ds(16 * c8, 16)]
                plsc.store_scatter(rp1, [db + chunk16[c8]], v)

        @plsc.parallel_loop(0, D)
        def _(j):
            jb = jnp.full((16,), 0, jnp.int32) + 2 * j
            for p8 in range(8):
                v = plsc.load_gather(rp1, [iota_rp_off[p8] + jb])
                stage[q, j, pl.ds(16 * p8, 16)] = v

    def substep(u, p):
        @pl.when(u < n_full)
        def _():
            wait_in(p)

            @pl.when(u + 1 < n_full)
            def _():
                fire_in(base + u + 1, 1 - p)

            @pl.when(u >= 2)
            def _():
                wait_out(p)

            transpose_block(p, p)
            row0 = pl.multiple_of((base + u) * D, D)
            pltpu.async_copy(
                stage.at[p], scr_hbm.at[pl.ds(row0, D), :], osems[p]
            )

    def step(k, _):
        substep(2 * k, 0)
        substep(2 * k + 1, 1)
        return 0

    fire_in(base, 0)
    lax.fori_loop(0, K1_STEPS, step, 0, unroll=False)
    wait_out(0)
    wait_out(1)


def _gather_body(
    idx_hbm, scr_hbm, out_hbm, idx_u, idx2_u, par_u, rows, rp2, stage, *sems
):
    isems = sems[:2]
    gsems = sems[2:4]
    osem = sems[4]
    wid = lax.axis_index("s") * NC + lax.axis_index("c")
    bofs = pl.multiple_of(wid * 128, 128)
    iota = lax.iota(jnp.int32, 16)
    chunk16 = [iota + 16 * c for c in range(8)]
    iota_rp = [(16 * p + iota) * RP for p in range(8)]

    def fire_idx(t, b):
        off = pl.multiple_of(t * 8, 8)
        pltpu.async_copy(
            idx_hbm.at[pl.ds(off, 8), pl.ds(bofs, 128)], idx_u.at[b],
            isems[b],
        )

    def wait_idx(b):
        pltpu.make_async_copy(
            idx_hbm.at[pl.ds(0, 8), pl.ds(0, 128)], idx_u.at[b], isems[b]
        ).wait()

    def compute_idx2(b):
        @plsc.parallel_loop(0, 8)
        def _(j):
            for c8 in range(8):
                v = idx_u[b, j, pl.ds(16 * c8, 16)]
                idx2_u[b, j, pl.ds(16 * c8, 16)] = v >> 1
                par_u[b, j, pl.ds(16 * c8, 16)] = (v & 1) * 64

    def fire_gather(ib, row, p):
        pltpu.async_copy(
            scr_hbm.at[idx2_u.at[ib, row]], rows.at[p], gsems[p]
        )

    def wait_gather(p):
        pltpu.make_async_copy(
            scr_hbm.at[pl.ds(0, 128)], rows.at[p], gsems[p]
        ).wait()

    def wait_out():
        pltpu.make_async_copy(
            stage, out_hbm.at[:, pl.ds(0, 8), pl.ds(0, 128)], osem
        ).wait()

    def transpose_l(j, p, it):
        @plsc.parallel_loop(0, 128)
        def _(r):
            rb = jnp.full((16,), 0, jnp.int32) + r * RP
            for c8 in range(8):
                v = rows[p, r, pl.ds(16 * c8, 16)]
                plsc.store_scatter(rp2, [rb + chunk16[c8]], v)

        parvs = [
            par_u[it, j, pl.ds(16 * p8, 16)] + iota_rp[p8] for p8 in range(8)
        ]

        @plsc.parallel_loop(0, D)
        def _(d):
            db = jnp.full((16,), 0, jnp.int32) + d
            for p8 in range(8):
                v = plsc.load_gather(rp2, [parvs[p8] + db])
                stage[d, j, pl.ds(16 * p8, 16)] = v

    def substep(t, it):
        @pl.when(t < NT)
        def _():
            @pl.when(t + 1 < NT)
            def _():
                wait_idx(1 - it)
                compute_idx2(1 - it)

            @pl.when(t + 2 < NT)
            def _():
                fire_idx(t + 2, it)

            @pl.when(t >= 1)
            def _():
                wait_out()

            for j in range(8):
                wait_gather(j % 2)
                transpose_l(j, j % 2, it)
                g2 = 8 * t + j + 2

                @pl.when(g2 < L)
                def _():
                    if j < 6:
                        fire_gather(it, j + 2, j % 2)
                    else:
                        fire_gather(1 - it, j - 6, j % 2)

            t8 = pl.multiple_of(t * 8, 8)
            pltpu.async_copy(
                stage, out_hbm.at[:, pl.ds(t8, 8), pl.ds(bofs, 128)], osem
            )

    def step(k, _):
        substep(2 * k, 0)
        substep(2 * k + 1, 1)
        return 0

    fire_idx(0, 0)
    wait_idx(0)
    compute_idx2(0)
    fire_idx(1, 1)
    fire_gather(0, 0, 0)
    fire_gather(0, 1, 1)
    lax.fori_loop(0, (NT + 1) // 2, step, 0, unroll=False)
    wait_out()


@functools.partial(jax.jit, static_argnames=())
def kernel(input, table):
    idx_t = input.astype(jnp.int32).T
    tbl_t = table.T
    cp = pltpu.CompilerParams(
        needs_layout_passes=False, disable_bounds_checks=True
    )
    mesh = plsc.VectorSubcoreMesh(core_axis_name="c", subcore_axis_name="s")

    relayout = pl.kernel(
        _relayout_body,
        out_type=jax.ShapeDtypeStruct((SROWS, 128), jnp.float32),
        mesh=mesh,
        compiler_params=cp,
        scratch_types=[
            pltpu.VMEM((2, D, 128), jnp.float32),
            pltpu.VMEM((D * RP,), jnp.float32),
            pltpu.VMEM((2, D, 128), jnp.float32),
        ]
        + [pltpu.SemaphoreType.DMA] * 4,
    )
    scratch = relayout(tbl_t)

    gather = pl.kernel(
        _gather_body,
        out_type=jax.ShapeDtypeStruct((D, L, B), jnp.float32),
        mesh=mesh,
        compiler_params=cp,
        scratch_types=[
            pltpu.VMEM((2, 8, 128), jnp.int32),
            pltpu.VMEM((2, 8, 128), jnp.int32),
            pltpu.VMEM((2, 8, 128), jnp.int32),
            pltpu.VMEM((2, 128, 128), jnp.float32),
            pltpu.VMEM((128 * RP,), jnp.float32),
            pltpu.VMEM((D, 8, 128), jnp.float32),
        ]
        + [pltpu.SemaphoreType.DMA] * 5,
    )
    out_t = gather(idx_t, scratch)
    return jnp.transpose(out_t, (2, 0, 1))


# speedup vs baseline: 4.4468x; 1.0404x over previous
import functools

import jax
import jax.numpy as jnp
from jax import lax
from jax.experimental import pallas as pl
from jax.experimental.pallas import tpu as pltpu
from jax.experimental.pallas import tpu_sc as plsc

B = 4096
L = 200
D = 64
V = 1000000
NC = 2
NS = 16
NW = NC * NS
RP = 129
VBT = (V + 127) // 128
VB_PER_W = VBT // NW
VB_EXTRA = VBT % NW
SROWS = VBT * 64
K1_STEPS = (VB_PER_W + 2) // 2
NT = L // 8


def _relayout_body(tbl_hbm, scr_hbm, blk, rp1, stage, *sems):
    gsems = sems[:2]
    osems = sems[2:]
    wid = lax.axis_index("s") * NC + lax.axis_index("c")
    iota = lax.iota(jnp.int32, 16)
    chunk16 = [iota + 16 * c for c in range(8)]
    iota_rp_off = [(16 * (p % 4) + iota) * RP + (p // 4) for p in range(8)]

    n_full = VB_PER_W + jnp.where(wid < VB_EXTRA, 1, 0)
    base = wid * VB_PER_W + jnp.minimum(wid, VB_EXTRA)

    def fire_in(u, p):
        off = pl.multiple_of(u * 128, 128)
        pltpu.async_copy(tbl_hbm.at[:, pl.ds(off, 128)], blk.at[p], gsems[p])

    def wait_in(p):
        pltpu.make_async_copy(
            tbl_hbm.at[:, pl.ds(0, 128)], blk.at[p], gsems[p]
        ).wait()

    def wait_out(q):
        pltpu.make_async_copy(
            stage.at[q], scr_hbm.at[pl.ds(0, D), :], osems[q]
        ).wait()

    def transpose_block(p, q):
        @plsc.parallel_loop(0, D)
        def _(d):
            db = jnp.full((16,), 0, jnp.int32) + d * RP
            for c8 in range(8):
                v = blk[p, d, pl.ds(16 * c8, 16)]
                plsc.store_scatter(rp1, [db + chunk16[c8]], v)

        @plsc.parallel_loop(0, D)
        def _(j):
            jb = jnp.full((16,), 0, jnp.int32) + 2 * j
            for p8 in range(8):
                v = plsc.load_gather(rp1, [iota_rp_off[p8] + jb])
                stage[q, j, pl.ds(16 * p8, 16)] = v

    def substep(u, p):
        @pl.when(u < n_full)
        def _():
            wait_in(p)

            @pl.when(u + 1 < n_full)
            def _():
                fire_in(base + u + 1, 1 - p)

            @pl.when(u >= 2)
            def _():
                wait_out(p)

            transpose_block(p, p)
            row0 = pl.multiple_of((base + u) * D, D)
            pltpu.async_copy(
                stage.at[p], scr_hbm.at[pl.ds(row0, D), :], osems[p]
            )

    def step(k, _):
        substep(2 * k, 0)
        substep(2 * k + 1, 1)
        return 0

    fire_in(base, 0)
    lax.fori_loop(0, K1_STEPS, step, 0, unroll=False)
    wait_out(0)
    wait_out(1)


def _gather_body(
    idx_hbm, scr_hbm, out_hbm, idx_u, idx2_u, par_u, rows, stage, *sems
):
    isems = sems[:2]
    gsems = sems[2:4]
    osem = sems[4]
    wid = lax.axis_index("s") * NC + lax.axis_index("c")
    bofs = pl.multiple_of(wid * 128, 128)
    iota = lax.iota(jnp.int32, 16)

    def fire_idx(t, b):
        off = pl.multiple_of(t * 8, 8)
        pltpu.async_copy(
            idx_hbm.at[pl.ds(off, 8), pl.ds(bofs, 128)], idx_u.at[b],
            isems[b],
        )

    def wait_idx(b):
        pltpu.make_async_copy(
            idx_hbm.at[pl.ds(0, 8), pl.ds(0, 128)], idx_u.at[b], isems[b]
        ).wait()

    def compute_idx2(b):
        @plsc.parallel_loop(0, 8)
        def _(j):
            for c8 in range(8):
                v = idx_u[b, j, pl.ds(16 * c8, 16)]
                idx2_u[b, j, pl.ds(16 * c8, 16)] = v >> 1
                par_u[b, j, pl.ds(16 * c8, 16)] = (v & 1) * 64

    def fire_gather(ib, row, p):
        pltpu.async_copy(
            scr_hbm.at[idx2_u.at[ib, row]], rows.at[p], gsems[p]
        )

    def wait_gather(p):
        pltpu.make_async_copy(
            scr_hbm.at[pl.ds(0, 128)], rows.at[p], gsems[p]
        ).wait()

    def wait_out():
        pltpu.make_async_copy(
            stage, out_hbm.at[:, pl.ds(0, 8), pl.ds(0, 128)], osem
        ).wait()

    def transpose_l(j, p, it):
        jv = jnp.full((16,), j, jnp.int32)

        @plsc.parallel_loop(0, 128)
        def _(pp):
            p16 = pp & 112
            rvec = iota + p16
            parv = par_u[it, j, pl.ds(p16, 16)]
            rotv = (iota + pp) & 15
            pr = parv + rotv
            for dq in range(4):
                dv = rotv + 16 * dq
                cv = pr + 16 * dq
                v = plsc.load_gather(rows.at[p], [rvec, cv])
                plsc.store_scatter(stage, [dv, jv, rvec], v)

    def substep(t, it):
        @pl.when(t < NT)
        def _():
            @pl.when(t + 1 < NT)
            def _():
                wait_idx(1 - it)
                compute_idx2(1 - it)

            @pl.when(t + 2 < NT)
            def _():
                fire_idx(t + 2, it)

            @pl.when(t >= 1)
            def _():
                wait_out()

            for j in range(8):
                wait_gather(j % 2)
                transpose_l(j, j % 2, it)
                g2 = 8 * t + j + 2

                @pl.when(g2 < L)
                def _():
                    if j < 6:
                        fire_gather(it, j + 2, j % 2)
                    else:
                        fire_gather(1 - it, j - 6, j % 2)

            t8 = pl.multiple_of(t * 8, 8)
            pltpu.async_copy(
                stage, out_hbm.at[:, pl.ds(t8, 8), pl.ds(bofs, 128)], osem
            )

    def step(k, _):
        substep(2 * k, 0)
        substep(2 * k + 1, 1)
        return 0

    fire_idx(0, 0)
    wait_idx(0)
    compute_idx2(0)
    fire_idx(1, 1)
    fire_gather(0, 0, 0)
    fire_gather(0, 1, 1)
    lax.fori_loop(0, (NT + 1) // 2, step, 0, unroll=False)
    wait_out()


@functools.partial(jax.jit, static_argnames=())
def kernel(input, table):
    idx_t = input.astype(jnp.int32).T
    tbl_t = table.T
    cp = pltpu.CompilerParams(
        needs_layout_passes=False, disable_bounds_checks=True
    )
    mesh = plsc.VectorSubcoreMesh(core_axis_name="c", subcore_axis_name="s")

    relayout = pl.kernel(
        _relayout_body,
        out_type=jax.ShapeDtypeStruct((SROWS, 128), jnp.float32),
        mesh=mesh,
        compiler_params=cp,
        scratch_types=[
            pltpu.VMEM((2, D, 128), jnp.float32),
            pltpu.VMEM((D * RP,), jnp.float32),
            pltpu.VMEM((2, D, 128), jnp.float32),
        ]
        + [pltpu.SemaphoreType.DMA] * 4,
    )
    scratch = relayout(tbl_t)

    gather = pl.kernel(
        _gather_body,
        out_type=jax.ShapeDtypeStruct((D, L, B), jnp.float32),
        mesh=mesh,
        compiler_params=cp,
        scratch_types=[
            pltpu.VMEM((2, 8, 128), jnp.int32),
            pltpu.VMEM((2, 8, 128), jnp.int32),
            pltpu.VMEM((2, 8, 128), jnp.int32),
            pltpu.VMEM((2, 128, 128), jnp.float32),
            pltpu.VMEM((D, 8, 128), jnp.float32),
        ]
        + [pltpu.SemaphoreType.DMA] * 5,
    )
    out_t = gather(idx_t, scratch)
    return jnp.transpose(out_t, (2, 0, 1))


# speedup vs baseline: 5.4348x; 1.2222x over previous
import functools

import jax
import jax.numpy as jnp
from jax import lax
from jax.experimental import pallas as pl
from jax.experimental.pallas import tpu as pltpu
from jax.experimental.pallas import tpu_sc as plsc

B = 4096
L = 200
D = 64
V = 1000000
NC = 2
NS = 16
NW = NC * NS
RP = 129
VBT = (V + 127) // 128
VB_PER_W = VBT // NW
VB_EXTRA = VBT % NW
SROWS = VBT * 64
K1_STEPS = (VB_PER_W + 4) // 4
NT = L // 8


def _relayout_body(tbl_hbm, scr_hbm, blk, rp1, stage, *sems):
    gsems = sems[:4]
    osems = sems[4:]
    wid = lax.axis_index("s") * NC + lax.axis_index("c")
    iota = lax.iota(jnp.int32, 16)
    chunk16 = [iota + 16 * c for c in range(8)]
    iota_rp_off = [(16 * (p % 4) + iota) * RP + (p // 4) for p in range(8)]

    n_full = VB_PER_W + jnp.where(wid < VB_EXTRA, 1, 0)
    base = wid * VB_PER_W + jnp.minimum(wid, VB_EXTRA)

    def fire_in(u, p):
        off = pl.multiple_of(u * 128, 128)
        pltpu.async_copy(tbl_hbm.at[:, pl.ds(off, 128)], blk.at[p], gsems[p])

    def wait_in(p):
        pltpu.make_async_copy(
            tbl_hbm.at[:, pl.ds(0, 128)], blk.at[p], gsems[p]
        ).wait()

    def wait_out(q):
        pltpu.make_async_copy(
            stage.at[q], scr_hbm.at[pl.ds(0, D), :], osems[q]
        ).wait()

    def transpose_block(p, q):
        @plsc.parallel_loop(0, D)
        def _(d):
            db = jnp.full((16,), 0, jnp.int32) + d * RP
            for c8 in range(8):
                v = blk[p, d, pl.ds(16 * c8, 16)]
                plsc.store_scatter(rp1, [db + chunk16[c8]], v)

        @plsc.parallel_loop(0, D)
        def _(j):
            jb = jnp.full((16,), 0, jnp.int32) + 2 * j
            for p8 in range(8):
                v = plsc.load_gather(rp1, [iota_rp_off[p8] + jb])
                stage[q, j, pl.ds(16 * p8, 16)] = v

    def substep(u, s4, s2):
        @pl.when(u < n_full)
        def _():
            wait_in(s4)

            @pl.when(u + 3 < n_full)
            def _():
                fire_in(base + u + 3, (s4 + 3) % 4)

            @pl.when(u >= 2)
            def _():
                wait_out(s2)

            transpose_block(s4, s2)
            row0 = pl.multiple_of((base + u) * D, D)
            pltpu.async_copy(
                stage.at[s2], scr_hbm.at[pl.ds(row0, D), :], osems[s2]
            )

    def step(k, _):
        for s in range(4):
            substep(4 * k + s, s, s % 2)
        return 0

    fire_in(base, 0)
    fire_in(base + 1, 1)
    fire_in(base + 2, 2)
    lax.fori_loop(0, K1_STEPS, step, 0, unroll=False)
    wait_out(0)
    wait_out(1)


def _gather_body(
    idx_hbm, scr_hbm, out_hbm, idx_u, idx2_u, par_u, rows, stage, *sems
):
    isems = sems[:3]
    gsems = sems[3:6]
    osem = sems[6]
    wid = lax.axis_index("s") * NC + lax.axis_index("c")
    bofs = pl.multiple_of(wid * 128, 128)
    iota = lax.iota(jnp.int32, 16)

    def fire_idx(t, b):
        off = pl.multiple_of(t * 8, 8)
        pltpu.async_copy(
            idx_hbm.at[pl.ds(off, 8), pl.ds(bofs, 128)], idx_u.at[b],
            isems[b],
        )

    def wait_idx(b):
        pltpu.make_async_copy(
            idx_hbm.at[pl.ds(0, 8), pl.ds(0, 128)], idx_u.at[b], isems[b]
        ).wait()

    def compute_idx2(b):
        @plsc.parallel_loop(0, 8)
        def _(j):
            for c8 in range(8):
                v = idx_u[b, j, pl.ds(16 * c8, 16)]
                idx2_u[b, j, pl.ds(16 * c8, 16)] = v >> 1
                par_u[b, j, pl.ds(16 * c8, 16)] = (v & 1) * 64

    def fire_gather(ib, row, p):
        pltpu.async_copy(
            scr_hbm.at[idx2_u.at[ib, row]], rows.at[p], gsems[p]
        )

    def wait_gather(p):
        pltpu.make_async_copy(
            scr_hbm.at[pl.ds(0, 128)], rows.at[p], gsems[p]
        ).wait()

    def wait_out():
        pltpu.make_async_copy(
            stage, out_hbm.at[:, pl.ds(0, 8), pl.ds(0, 128)], osem
        ).wait()

    def transpose_l(j, p, it):
        jv = jnp.full((16,), j, jnp.int32)

        @plsc.parallel_loop(0, 128)
        def _(pp):
            p16 = pp & 112
            rvec = iota + p16
            parv = par_u[it, j, pl.ds(p16, 16)]
            rotv = (iota + pp) & 15
            pr = parv + rotv
            for dq in range(4):
                dv = rotv + 16 * dq
                cv = pr + 16 * dq
                v = plsc.load_gather(rows.at[p], [rvec, cv])
                plsc.store_scatter(stage, [dv, jv, rvec], v)

    def substep(t, st):
        @pl.when(t < NT)
        def _():
            @pl.when(t + 1 < NT)
            def _():
                wait_idx((st + 1) % 3)
                compute_idx2((st + 1) % 3)

            @pl.when(t + 2 < NT)
            def _():
                fire_idx(t + 2, (st + 2) % 3)

            @pl.when(t >= 1)
            def _():
                wait_out()

            for j in range(8):
                bg = (2 * st + j) % 3
                wait_gather(bg)
                transpose_l(j, bg, st)
                g2 = 8 * t + j + 2

                @pl.when(g2 < L)
                def _():
                    if j < 6:
                        fire_gather(st, j + 2, (bg + 2) % 3)
                    else:
                        fire_gather((st + 1) % 3, j - 6, (bg + 2) % 3)

            t8 = pl.multiple_of(t * 8, 8)
            pltpu.async_copy(
                stage, out_hbm.at[:, pl.ds(t8, 8), pl.ds(bofs, 128)], osem
            )

    def step(k, _):
        for s in range(3):
            substep(3 * k + s, s)
        return 0

    fire_idx(0, 0)
    wait_idx(0)
    compute_idx2(0)
    fire_idx(1, 1)
    fire_gather(0, 0, 0)
    fire_gather(0, 1, 1)
    lax.fori_loop(0, (NT + 2) // 3, step, 0, unroll=False)
    wait_out()


@functools.partial(jax.jit, static_argnames=())
def kernel(input, table):
    idx_t = input.astype(jnp.int32).T
    tbl_t = table.T
    cp = pltpu.CompilerParams(
        needs_layout_passes=False, disable_bounds_checks=True
    )
    mesh = plsc.VectorSubcoreMesh(core_axis_name="c", subcore_axis_name="s")

    relayout = pl.kernel(
        _relayout_body,
        out_type=jax.ShapeDtypeStruct((SROWS, 128), jnp.float32),
        mesh=mesh,
        compiler_params=cp,
        scratch_types=[
            pltpu.VMEM((4, D, 128), jnp.float32),
            pltpu.VMEM((D * RP,), jnp.float32),
            pltpu.VMEM((2, D, 128), jnp.float32),
        ]
        + [pltpu.SemaphoreType.DMA] * 6,
    )
    scratch = relayout(tbl_t)

    gather = pl.kernel(
        _gather_body,
        out_type=jax.ShapeDtypeStruct((D, L, B), jnp.float32),
        mesh=mesh,
        compiler_params=cp,
        scratch_types=[
            pltpu.VMEM((3, 8, 128), jnp.int32),
            pltpu.VMEM((3, 8, 128), jnp.int32),
            pltpu.VMEM((3, 8, 128), jnp.int32),
            pltpu.VMEM((3, 128, 128), jnp.float32),
            pltpu.VMEM((D, 8, 128), jnp.float32),
        ]
        + [pltpu.SemaphoreType.DMA] * 7,
    )
    out_t = gather(idx_t, scratch)
    return jnp.transpose(out_t, (2, 0, 1))
